# baseline (speedup 1.0000x reference)
import functools

import jax
import jax.numpy as jnp
from jax import lax
from jax.experimental import pallas as pl
from jax.experimental.pallas import tpu as pltpu
from jax.experimental.pallas import tpu_sc as plsc

_BATCH = 16384
_D = 32
_NC = 2
_NS = 16
_NW = _NC * _NS
_BPW = _BATCH // _NW
_CHUNK = 128
_NCHUNK = _BPW // _CHUNK
_L = 16


def _mf_body(uid_hbm, iid_hbm, p_hbm, q_hbm, ub_hbm, ib_hbm, out_hbm,
             uidx, iidx, prow, qrow, bu_v, bi_v, out_v, sem):
    wid = lax.axis_index("s") * _NC + lax.axis_index("c")
    pltpu.sync_copy(uid_hbm.at[wid], uidx)
    pltpu.sync_copy(iid_hbm.at[wid], iidx)

    cps = []
    for c in range(_NCHUNK):
        sl = pl.ds(c * _CHUNK, _CHUNK)
        cps.append(pltpu.async_copy(p_hbm.at[uidx.at[c]], prow.at[sl], sem))
        cps.append(pltpu.async_copy(q_hbm.at[iidx.at[c]], qrow.at[sl], sem))
        cps.append(pltpu.async_copy(ub_hbm.at[uidx.at[c]], bu_v.at[sl], sem))
        cps.append(pltpu.async_copy(ib_hbm.at[iidx.at[c]], bi_v.at[sl], sem))
    for cp in cps:
        cp.wait()

    lane = lax.broadcasted_iota(jnp.int32, (_L,), 0)

    def group(g, carry):
        base = g * _L
        acc = bu_v[pl.ds(base, _L)] + bi_v[pl.ds(base, _L)]
        row_idx = lane + base
        for k in range(_D):
            col = jnp.full((_L,), k, jnp.int32)
            pv = plsc.load_gather(prow, [row_idx, col])
            qv = plsc.load_gather(qrow, [row_idx, col])
            acc = acc + pv * qv
        out_v[pl.ds(base, _L)] = acc
        return carry

    lax.fori_loop(0, _BPW // _L, group, 0)

    pltpu.sync_copy(out_v, out_hbm.at[pl.ds(wid * _BPW, _BPW)])


@jax.jit
def _mf(uid3, iid3, P, Q, ub, ib):
    mesh = plsc.VectorSubcoreMesh(core_axis_name="c", subcore_axis_name="s")
    return pl.kernel(
        _mf_body,
        mesh=mesh,
        compiler_params=pltpu.CompilerParams(
            needs_layout_passes=False, use_tc_tiling_on_sc=False),
        out_type=jax.ShapeDtypeStruct((_BATCH,), jnp.float32),
        scratch_types=[
            pltpu.VMEM((_NCHUNK, _CHUNK), jnp.int32),
            pltpu.VMEM((_NCHUNK, _CHUNK), jnp.int32),
            pltpu.VMEM((_BPW, _D), jnp.float32),
            pltpu.VMEM((_BPW, _D), jnp.float32),
            pltpu.VMEM((_BPW,), jnp.float32),
            pltpu.VMEM((_BPW,), jnp.float32),
            pltpu.VMEM((_BPW,), jnp.float32),
            pltpu.SemaphoreType.DMA,
        ],
    )(uid3, iid3, P, Q, ub, ib)


def kernel(user_id, item_id, P, Q, user_bias, item_bias):
    uid3 = user_id.reshape(_NW, _NCHUNK, _CHUNK)
    iid3 = item_id.reshape(_NW, _NCHUNK, _CHUNK)
    ub = user_bias.reshape(-1)
    ib = item_bias.reshape(-1)
    return _mf(uid3, iid3, P, Q, ub, ib)
